# baseline (speedup 1.0000x reference)
import functools

import jax
import jax.numpy as jnp
from jax import lax
from jax.experimental import pallas as pl
from jax.experimental.pallas import tpu as pltpu
from jax.experimental.pallas import tpu_sc as plsc

D = 128
NUM_CORES = 2
NUM_SUBCORES = 16
NUM_TILES = NUM_CORES * NUM_SUBCORES
CHUNK = 80
GROUPS = CHUNK // 16


def _uv_body(x_ref, w1_ref, b1_ref, u_ref, v_ref):
    xb = x_ref[...]
    w1 = w1_ref[...]
    dn = (((1,), (1,)), ((), ()))
    u = lax.dot_general(xb, w1[:, :D], dn, preferred_element_type=jnp.float32)
    v = lax.dot_general(xb, w1[:, D:], dn, preferred_element_type=jnp.float32)
    u_ref[...] = (u + b1_ref[...]).astype(jnp.bfloat16)
    v_ref[...] = v.astype(jnp.bfloat16)


def _compute_uv(x, W1, b1):
    n = x.shape[0]
    blk = 1000
    grid = (n // blk,)
    return pl.pallas_call(
        _uv_body,
        grid=grid,
        in_specs=[
            pl.BlockSpec((blk, D), lambda i: (i, 0)),
            pl.BlockSpec((D, 2 * D), lambda i: (0, 0)),
            pl.BlockSpec((1, D), lambda i: (0, 0)),
        ],
        out_specs=[
            pl.BlockSpec((blk, D), lambda i: (i, 0)),
            pl.BlockSpec((blk, D), lambda i: (i, 0)),
        ],
        out_shape=[
            jax.ShapeDtypeStruct((n, D), jnp.bfloat16),
            jax.ShapeDtypeStruct((n, D), jnp.bfloat16),
        ],
    )(x, W1, b1.reshape(1, D))


def _make_edge_kernel(num_edges):
    ept = num_edges // NUM_TILES
    chunks = ept // CHUNK
    mesh = plsc.VectorSubcoreMesh(
        core_axis_name="c", subcore_axis_name="s",
        num_cores=NUM_CORES, num_subcores=NUM_SUBCORES)

    @functools.partial(
        pl.kernel,
        out_type=jax.ShapeDtypeStruct((num_edges,), jnp.float32),
        mesh=mesh,
        compiler_params=pltpu.CompilerParams(
            needs_layout_passes=False, use_tc_tiling_on_sc=False),
        scratch_types=[
            pltpu.VMEM((ept,), jnp.int32),
            pltpu.VMEM((ept,), jnp.int32),
            pltpu.VMEM((ept,), jnp.float32),
            pltpu.VMEM((ept,), jnp.float32),
            pltpu.VMEM((CHUNK, D // 2), jnp.float32),
            pltpu.VMEM((CHUNK, D // 2), jnp.float32),
            pltpu.VMEM((CHUNK, D // 2), jnp.float32),
            pltpu.VMEM((CHUNK, D // 2), jnp.float32),
            pltpu.VMEM((D // 2,), jnp.float32),
            pltpu.VMEM((16,), jnp.float32),
            pltpu.VMEM((256,), jnp.float32),
            pltpu.SemaphoreType.DMA,
            pltpu.SemaphoreType.DMA,
            pltpu.SemaphoreType.DMA,
            pltpu.SemaphoreType.DMA,
        ],
    )
    def edge_kernel(u_hbm, v_hbm, row_hbm, col_hbm, ev_hbm, w2_hbm, b2_hbm,
                    out_hbm, idx_r, idx_c, ev_v, out_v, ru0, rv0, ru1, rv1,
                    w2_v, b2_v, pbuf, sem_u0, sem_v0, sem_u1, sem_v1):
        wid = lax.axis_index("s") * NUM_CORES + lax.axis_index("c")
        tile_base = wid * ept
        pltpu.sync_copy(row_hbm.at[pl.ds(tile_base, ept)], idx_r)
        pltpu.sync_copy(col_hbm.at[pl.ds(tile_base, ept)], idx_c)
        pltpu.sync_copy(ev_hbm.at[pl.ds(tile_base, ept)], ev_v)
        pltpu.sync_copy(w2_hbm, w2_v)
        pltpu.sync_copy(b2_hbm, b2_v)
        b2vec = b2_v[...]
        w2_regs = [plsc.bitcast(w2_v[pl.ds(k * 16, 16)], jnp.bfloat16)
                   for k in range(D // 32)]
        lane_iota = lax.iota(jnp.int32, 16)

        bufs = ((ru0, rv0, sem_u0, sem_v0), (ru1, rv1, sem_u1, sem_v1))

        def issue(i, b):
            ru, rv, sem_u, sem_v = bufs[b]
            pltpu.async_copy(u_hbm.at[idx_r.at[pl.ds(i * CHUNK, CHUNK)]],
                             ru, sem_u)
            pltpu.async_copy(v_hbm.at[idx_c.at[pl.ds(i * CHUNK, CHUNK)]],
                             rv, sem_v)

        def wait(b):
            ru, rv, sem_u, sem_v = bufs[b]
            pltpu.make_async_copy(u_hbm.at[idx_r.at[pl.ds(0, CHUNK)]],
                                  ru, sem_u).wait()
            pltpu.make_async_copy(v_hbm.at[idx_c.at[pl.ds(0, CHUNK)]],
                                  rv, sem_v).wait()

        def compute(i, b):
            ru, rv, _, _ = bufs[b]
            cbase = i * CHUNK

            def group_body(g, gcarry):
                partials = []
                for j in range(16):
                    e = g * 16 + j
                    ts = []
                    for k in range(D // 32):
                        uvals = plsc.bitcast(ru[e, pl.ds(k * 16, 16)],
                                             jnp.bfloat16)
                        vvals = plsc.bitcast(rv[e, pl.ds(k * 16, 16)],
                                             jnp.bfloat16)
                        t = jnp.maximum(uvals + vvals, jnp.bfloat16(0.0))
                        ts.append(t * w2_regs[k])
                    tsum = (ts[0] + ts[1]) + (ts[2] + ts[3])
                    ta, tb = plsc.unpack(
                        tsum, format=plsc.PackFormat.INTERLEAVED,
                        preferred_element_type=jnp.float32)
                    partials.append(ta + tb)
                for j in range(16):
                    pbuf[pl.ds(j * 16, 16)] = partials[j]
                cols = [plsc.load_gather(pbuf, [lane16]) for lane16 in
                        [lane_iota * 16 + c for c in range(16)]]
                while len(cols) > 1:
                    cols = [a + b for a, b in zip(cols[::2], cols[1::2])]
                s_vec = cols[0]
                att = 1.0 / (1.0 + jnp.exp(-(s_vec + b2vec)))
                evg = ev_v[pl.ds(cbase + g * 16, 16)]
                out_v[pl.ds(cbase + g * 16, 16)] = att * evg
                return gcarry

            lax.fori_loop(0, GROUPS, group_body, 0)

        issue(0, 0)

        def pair_body(t, carry):
            i0 = t * 2
            issue(i0 + 1, 1)
            wait(0)
            compute(i0, 0)
            issue(i0 + 2, 0)
            wait(1)
            compute(i0 + 1, 1)
            return carry

        lax.fori_loop(0, chunks // 2, pair_body, 0)
        wait(0)
        compute(chunks - 1, 0)

        pltpu.sync_copy(out_v, out_hbm.at[pl.ds(tile_base, ept)])

    return edge_kernel


@jax.jit
def kernel(x, edge_index, edge_values, W1, b1, W2, b2):
    row = edge_index[0]
    col = edge_index[1]
    u, v = _compute_uv(x, W1, b1)
    n = x.shape[0]
    u = jax.lax.bitcast_convert_type(u.reshape(n, D // 2, 2), jnp.float32)
    v = jax.lax.bitcast_convert_type(v.reshape(n, D // 2, 2), jnp.float32)
    w2 = jax.lax.bitcast_convert_type(
        W2[0].astype(jnp.bfloat16).reshape(D // 2, 2), jnp.float32)
    b2v = jnp.full((16,), b2[0], jnp.float32)
    edge_fn = _make_edge_kernel(edge_values.shape[0])
    return edge_fn(u, v, row, col, edge_values, w2, b2v)


# speedup vs baseline: 1.0713x; 1.0713x over previous
import functools

import jax
import jax.numpy as jnp
from jax import lax
from jax.experimental import pallas as pl
from jax.experimental.pallas import tpu as pltpu
from jax.experimental.pallas import tpu_sc as plsc

D = 128
NUM_CORES = 2
NUM_SUBCORES = 16
NUM_TILES = NUM_CORES * NUM_SUBCORES
CHUNK = 80
GROUPS = CHUNK // 16
NBUF = 5


def _uv_body(x_ref, w1_ref, b1_ref, u_ref, v_ref):
    xb = x_ref[...]
    w1 = w1_ref[...]
    dn = (((1,), (1,)), ((), ()))
    u = lax.dot_general(xb, w1[:, :D], dn, preferred_element_type=jnp.float32)
    v = lax.dot_general(xb, w1[:, D:], dn, preferred_element_type=jnp.float32)
    u_ref[...] = (u + b1_ref[...]).astype(jnp.bfloat16)
    v_ref[...] = v.astype(jnp.bfloat16)


def _compute_uv(x, W1, b1):
    n = x.shape[0]
    blk = 1000
    grid = (n // blk,)
    return pl.pallas_call(
        _uv_body,
        grid=grid,
        in_specs=[
            pl.BlockSpec((blk, D), lambda i: (i, 0)),
            pl.BlockSpec((D, 2 * D), lambda i: (0, 0)),
            pl.BlockSpec((1, D), lambda i: (0, 0)),
        ],
        out_specs=[
            pl.BlockSpec((blk, D), lambda i: (i, 0)),
            pl.BlockSpec((blk, D), lambda i: (i, 0)),
        ],
        out_shape=[
            jax.ShapeDtypeStruct((n, D), jnp.bfloat16),
            jax.ShapeDtypeStruct((n, D), jnp.bfloat16),
        ],
    )(x, W1, b1.reshape(1, D))


def _make_edge_kernel(num_edges):
    ept = num_edges // NUM_TILES
    chunks = ept // CHUNK
    mesh = plsc.VectorSubcoreMesh(
        core_axis_name="c", subcore_axis_name="s",
        num_cores=NUM_CORES, num_subcores=NUM_SUBCORES)

    @functools.partial(
        pl.kernel,
        out_type=jax.ShapeDtypeStruct((num_edges,), jnp.float32),
        mesh=mesh,
        compiler_params=pltpu.CompilerParams(
            needs_layout_passes=False, use_tc_tiling_on_sc=False),
        scratch_types=[
            pltpu.VMEM((ept,), jnp.int32),
            pltpu.VMEM((ept,), jnp.int32),
            pltpu.VMEM((ept,), jnp.float32),
            pltpu.VMEM((ept,), jnp.float32),
            [pltpu.VMEM((CHUNK, D // 2), jnp.float32)] * (2 * NBUF),
            pltpu.VMEM((D // 2,), jnp.float32),
            pltpu.VMEM((16,), jnp.float32),
            pltpu.VMEM((256,), jnp.float32),
            [pltpu.SemaphoreType.DMA] * (2 * NBUF),
        ],
    )
    def edge_kernel(u_hbm, v_hbm, row_hbm, col_hbm, ev_hbm, w2_hbm, b2_hbm,
                    out_hbm, idx_r, idx_c, ev_v, out_v, rbufs,
                    w2_v, b2_v, pbuf, sems):
        wid = lax.axis_index("s") * NUM_CORES + lax.axis_index("c")
        tile_base = wid * ept
        pltpu.sync_copy(row_hbm.at[pl.ds(tile_base, ept)], idx_r)
        pltpu.sync_copy(col_hbm.at[pl.ds(tile_base, ept)], idx_c)
        pltpu.sync_copy(ev_hbm.at[pl.ds(tile_base, ept)], ev_v)
        pltpu.sync_copy(w2_hbm, w2_v)
        pltpu.sync_copy(b2_hbm, b2_v)
        b2vec = b2_v[...]
        w2_regs = [plsc.bitcast(w2_v[pl.ds(k * 16, 16)], jnp.bfloat16)
                   for k in range(D // 32)]
        lane_iota = lax.iota(jnp.int32, 16)

        bufs = tuple(
            (rbufs[2 * b], rbufs[2 * b + 1], sems[2 * b], sems[2 * b + 1])
            for b in range(NBUF))

        def issue(i, b):
            ru, rv, sem_u, sem_v = bufs[b]
            pltpu.async_copy(u_hbm.at[idx_r.at[pl.ds(i * CHUNK, CHUNK)]],
                             ru, sem_u)
            pltpu.async_copy(v_hbm.at[idx_c.at[pl.ds(i * CHUNK, CHUNK)]],
                             rv, sem_v)

        def wait(b):
            ru, rv, sem_u, sem_v = bufs[b]
            pltpu.make_async_copy(u_hbm.at[idx_r.at[pl.ds(0, CHUNK)]],
                                  ru, sem_u).wait()
            pltpu.make_async_copy(v_hbm.at[idx_c.at[pl.ds(0, CHUNK)]],
                                  rv, sem_v).wait()

        def compute(i, b):
            ru, rv, _, _ = bufs[b]
            cbase = i * CHUNK

            def group_body(g, gcarry):
                partials = []
                for j in range(16):
                    e = g * 16 + j
                    ts = []
                    for k in range(D // 32):
                        uvals = plsc.bitcast(ru[e, pl.ds(k * 16, 16)],
                                             jnp.bfloat16)
                        vvals = plsc.bitcast(rv[e, pl.ds(k * 16, 16)],
                                             jnp.bfloat16)
                        t = jnp.maximum(uvals + vvals, jnp.bfloat16(0.0))
                        ts.append(t * w2_regs[k])
                    tsum = (ts[0] + ts[1]) + (ts[2] + ts[3])
                    ta, tb = plsc.unpack(
                        tsum, format=plsc.PackFormat.INTERLEAVED,
                        preferred_element_type=jnp.float32)
                    partials.append(ta + tb)
                for j in range(16):
                    pbuf[pl.ds(j * 16, 16)] = partials[j]
                cols = [plsc.load_gather(pbuf, [lane16]) for lane16 in
                        [lane_iota * 16 + c for c in range(16)]]
                while len(cols) > 1:
                    cols = [a + b for a, b in zip(cols[::2], cols[1::2])]
                s_vec = cols[0]
                att = 1.0 / (1.0 + jnp.exp(-(s_vec + b2vec)))
                evg = ev_v[pl.ds(cbase + g * 16, 16)]
                out_v[pl.ds(cbase + g * 16, 16)] = att * evg
                return gcarry

            lax.fori_loop(0, GROUPS, group_body, 0)

        for b in range(NBUF - 1):
            issue(b, b)

        def round_body(t, carry):
            base_i = t * NBUF
            for b in range(NBUF):
                i = base_i + b
                wait(b)
                nxt = i + NBUF - 1

                @pl.when(nxt < chunks)
                def _():
                    issue(nxt, (b + NBUF - 1) % NBUF)

                compute(i, b)
            return carry

        lax.fori_loop(0, chunks // NBUF, round_body, 0)

        pltpu.sync_copy(out_v, out_hbm.at[pl.ds(tile_base, ept)])

    return edge_kernel


@jax.jit
def kernel(x, edge_index, edge_values, W1, b1, W2, b2):
    row = edge_index[0]
    col = edge_index[1]
    u, v = _compute_uv(x, W1, b1)
    n = x.shape[0]
    u = jax.lax.bitcast_convert_type(u.reshape(n, D // 2, 2), jnp.float32)
    v = jax.lax.bitcast_convert_type(v.reshape(n, D // 2, 2), jnp.float32)
    w2 = jax.lax.bitcast_convert_type(
        W2[0].astype(jnp.bfloat16).reshape(D // 2, 2), jnp.float32)
    b2v = jnp.full((16,), b2[0], jnp.float32)
    edge_fn = _make_edge_kernel(edge_values.shape[0])
    return edge_fn(u, v, row, col, edge_values, w2, b2v)
